# baseline (speedup 1.0000x reference)
import functools

import numpy as np

import jax
import jax.numpy as jnp
from jax.experimental import pallas as pl
from jax.experimental.pallas import tpu as pltpu

_PATCH = 3
_PAD = _PATCH // 2
_NUM_PAIRS = 4


def _loss_kernel(f8_ref, gm_ref, x_ref, y_ref, out_ref, *, H, W, TC, CSUB,
                 n_steps):
    step = pl.program_id(0) * pl.num_programs(1) + pl.program_id(1)

    @pl.when(step == 0)
    def _():
        out_ref[...] = jnp.zeros_like(out_ref)

    K = CSUB * H
    f8 = f8_ref[...]

    def chunk_Z(xc3, yc3, Z):
        xc = xc3.reshape(K, W)
        yc = yc3.reshape(K, W)
        rp = lambda a: pltpu.roll(a, W - 1, axis=1)
        rm = lambda a: pltpu.roll(a, 1, axis=1)

        def pair_duo(pxA, pyA, pxB, pyB, idx, Z):
            one = jnp.bfloat16(1.0)
            DX = jnp.concatenate([pxA - xc, pxB - xc],
                                 axis=1).astype(jnp.bfloat16)
            DY = jnp.concatenate([pyA - yc, pyB - yc],
                                 axis=1).astype(jnp.bfloat16)
            NX = DX * jax.lax.rsqrt(DX * DX + jnp.bfloat16(0.81))
            NY = DY * jax.lax.rsqrt(DY * DY + jnp.bfloat16(0.81))
            D = NX - NY
            R = one / (D * D + jnp.bfloat16(0.1))
            u = jnp.dot(f8, R, preferred_element_type=jnp.float32)
            return Z + u * gm_ref[idx]

        xd = pltpu.roll(xc, K - 1, axis=0)
        yd = pltpu.roll(yc, K - 1, axis=0)
        Z = pair_duo(rp(xc), rp(yc), xd, yd, 0, Z)
        Z = pair_duo(rm(xd), rm(yd), rp(xd), rp(yd), 1, Z)
        return Z

    n_chunks = TC // CSUB
    Z0 = jnp.zeros((8, 2 * W), jnp.float32)

    if n_chunks <= 4:
        Z = Z0
        for ci in range(n_chunks):
            Z = chunk_Z(x_ref[0, pl.ds(ci * CSUB, CSUB), :, :],
                        y_ref[0, pl.ds(ci * CSUB, CSUB), :, :], Z)
    else:
        def body(ci, Z):
            c0 = pl.multiple_of(ci * CSUB, CSUB)
            return chunk_Z(x_ref[0, pl.ds(c0, CSUB), :, :],
                           y_ref[0, pl.ds(c0, CSUB), :, :], Z)

        Z = jax.lax.fori_loop(0, n_chunks, body, Z0)
    out_ref[...] += jnp.sum(Z).reshape(1, 1)


@jax.jit
def _geometry_loss(x, y):
    B, C, H, W = x.shape
    HW = H * W

    xf = x.astype(jnp.float32)
    yf = y.astype(jnp.float32)

    TC = C
    for cand in (128, 64, 32, C):
        if C % cand == 0:
            TC = cand
            break
    n_ct = C // TC
    CSUB = 32 if TC % 32 == 0 else (8 if TC % 8 == 0 else TC)

    f = ((np.arange(H) >= _PAD) & (np.arange(H) < H - _PAD)).astype(np.float32)
    g = ((np.arange(W) >= _PAD) & (np.arange(W) < W - _PAD)).astype(np.float32)
    f1 = np.concatenate([f[1:], [0.0]]).astype(np.float32)
    g1p = np.roll(g, -1)
    g1m = np.roll(g, 1)

    K = CSUB * H
    F8 = np.zeros((8, K), np.float32)
    F8[0] = np.tile(f, CSUB)
    F8[1] = np.tile(f1, CSUB)

    Gm = np.zeros((2, 8, 2 * W), np.float32)
    Gm[0, 0, :W] = g + g1p
    Gm[0, 0, W:] = g
    Gm[0, 1, W:] = g
    Gm[1, 0, :W] = g
    Gm[1, 1, :W] = g1m
    Gm[1, 0, W:] = g
    Gm[1, 1, W:] = g1p

    pair_offsets = (1, W - 1, W, W + 1)
    interior = (f[:, None] * g[None, :]).reshape(HW)
    mask_sum = sum(float((interior + np.roll(interior, -s)).sum())
                   for s in pair_offsets)
    mask_const = mask_sum * B * C
    inv_norm = 1.0 / (C * _PATCH * _PATCH * B * H * W)

    kernel_body = functools.partial(
        _loss_kernel, H=H, W=W, TC=TC, CSUB=CSUB, n_steps=B * n_ct)

    out = pl.pallas_call(
        kernel_body,
        out_shape=jax.ShapeDtypeStruct((1, 1), jnp.float32),
        grid=(B, n_ct),
        in_specs=[
            pl.BlockSpec((8, K), lambda b, c: (0, 0)),
            pl.BlockSpec((2, 8, 2 * W), lambda b, c: (0, 0, 0)),
            pl.BlockSpec((1, TC, H, W), lambda b, c: (b, c, 0, 0)),
            pl.BlockSpec((1, TC, H, W), lambda b, c: (b, c, 0, 0)),
        ],
        out_specs=pl.BlockSpec((1, 1), lambda b, c: (0, 0)),
        compiler_params=pltpu.CompilerParams(
            dimension_semantics=("arbitrary", "arbitrary"),
            vmem_limit_bytes=int(40 << 20)),
    )(jnp.asarray(F8, jnp.bfloat16), jnp.asarray(Gm), xf, yf)

    return (mask_const - 0.1 * out[0, 0]) * inv_norm


def kernel(x, y):
    return _geometry_loss(x, y)


# speedup vs baseline: 1.0192x; 1.0192x over previous
import functools

import numpy as np

import jax
import jax.numpy as jnp
from jax.experimental import pallas as pl
from jax.experimental.pallas import tpu as pltpu

_PATCH = 3
_PAD = _PATCH // 2
_NUM_PAIRS = 4


def _loss_kernel(f8_ref, gm_ref, x_ref, y_ref, out_ref, *, H, W, TC, CSUB,
                 n_steps):
    step = pl.program_id(0) * pl.num_programs(1) + pl.program_id(1)

    @pl.when(step == 0)
    def _():
        out_ref[...] = jnp.zeros_like(out_ref)

    K = CSUB * H
    f8 = f8_ref[...]

    def chunk_Z(xc3, yc3, Z):
        xc = xc3.reshape(K, W)
        yc = yc3.reshape(K, W)
        rp = lambda a: pltpu.roll(a, W - 1, axis=1)
        rm = lambda a: pltpu.roll(a, 1, axis=1)

        def pair_duo(pxA, pyA, pxB, pyB, idx, Z):
            one = jnp.bfloat16(1.0)
            DX = jnp.concatenate([pxA - xc, pxB - xc],
                                 axis=1).astype(jnp.bfloat16)
            DY = jnp.concatenate([pyA - yc, pyB - yc],
                                 axis=1).astype(jnp.bfloat16)
            NX = DX * jax.lax.rsqrt(DX * DX + jnp.bfloat16(0.81))
            NY = DY * jax.lax.rsqrt(DY * DY + jnp.bfloat16(0.81))
            D = NX - NY
            R = one / (D * D + jnp.bfloat16(0.1))
            u = jnp.dot(f8, R, preferred_element_type=jnp.float32)
            return Z + u * gm_ref[idx]

        xd = pltpu.roll(xc, K - 1, axis=0)
        yd = pltpu.roll(yc, K - 1, axis=0)
        Z = pair_duo(rp(xc), rp(yc), xd, yd, 0, Z)
        Z = pair_duo(rm(xd), rm(yd), rp(xd), rp(yd), 1, Z)
        return Z

    n_chunks = TC // CSUB
    Z0 = jnp.zeros((8, 2 * W), jnp.float32)

    if n_chunks <= 4:
        Z = Z0
        for ci in range(n_chunks):
            Z = chunk_Z(x_ref[0, pl.ds(ci * CSUB, CSUB), :, :],
                        y_ref[0, pl.ds(ci * CSUB, CSUB), :, :], Z)
    else:
        def body(ci, Z):
            c0 = pl.multiple_of(ci * CSUB, CSUB)
            return chunk_Z(x_ref[0, pl.ds(c0, CSUB), :, :],
                           y_ref[0, pl.ds(c0, CSUB), :, :], Z)

        Z = jax.lax.fori_loop(0, n_chunks, body, Z0)
    out_ref[...] += jnp.sum(Z).reshape(1, 1)


@jax.jit
def _geometry_loss(x, y):
    B, C, H, W = x.shape
    HW = H * W

    xf = x.astype(jnp.float32)
    yf = y.astype(jnp.float32)

    TC = C
    for cand in (64, 32, C):
        if C % cand == 0:
            TC = cand
            break
    n_ct = C // TC
    CSUB = 32 if TC % 32 == 0 else (8 if TC % 8 == 0 else TC)

    f = ((np.arange(H) >= _PAD) & (np.arange(H) < H - _PAD)).astype(np.float32)
    g = ((np.arange(W) >= _PAD) & (np.arange(W) < W - _PAD)).astype(np.float32)
    f1 = np.concatenate([f[1:], [0.0]]).astype(np.float32)
    g1p = np.roll(g, -1)
    g1m = np.roll(g, 1)

    K = CSUB * H
    F8 = np.zeros((8, K), np.float32)
    F8[0] = np.tile(f, CSUB)
    F8[1] = np.tile(f1, CSUB)

    Gm = np.zeros((2, 8, 2 * W), np.float32)
    Gm[0, 0, :W] = g + g1p
    Gm[0, 0, W:] = g
    Gm[0, 1, W:] = g
    Gm[1, 0, :W] = g
    Gm[1, 1, :W] = g1m
    Gm[1, 0, W:] = g
    Gm[1, 1, W:] = g1p

    pair_offsets = (1, W - 1, W, W + 1)
    interior = (f[:, None] * g[None, :]).reshape(HW)
    mask_sum = sum(float((interior + np.roll(interior, -s)).sum())
                   for s in pair_offsets)
    mask_const = mask_sum * B * C
    inv_norm = 1.0 / (C * _PATCH * _PATCH * B * H * W)

    kernel_body = functools.partial(
        _loss_kernel, H=H, W=W, TC=TC, CSUB=CSUB, n_steps=B * n_ct)

    out = pl.pallas_call(
        kernel_body,
        out_shape=jax.ShapeDtypeStruct((1, 1), jnp.float32),
        grid=(B, n_ct),
        in_specs=[
            pl.BlockSpec((8, K), lambda b, c: (0, 0)),
            pl.BlockSpec((2, 8, 2 * W), lambda b, c: (0, 0, 0)),
            pl.BlockSpec((1, TC, H, W), lambda b, c: (b, c, 0, 0)),
            pl.BlockSpec((1, TC, H, W), lambda b, c: (b, c, 0, 0)),
        ],
        out_specs=pl.BlockSpec((1, 1), lambda b, c: (0, 0)),
        compiler_params=pltpu.CompilerParams(
            dimension_semantics=("arbitrary", "arbitrary"),
            vmem_limit_bytes=int(40 << 20)),
    )(jnp.asarray(F8, jnp.bfloat16), jnp.asarray(Gm), xf, yf)

    return (mask_const - 0.1 * out[0, 0]) * inv_norm


def kernel(x, y):
    return _geometry_loss(x, y)


# speedup vs baseline: 1.2240x; 1.2009x over previous
import functools

import numpy as np

import jax
import jax.numpy as jnp
from jax.experimental import pallas as pl
from jax.experimental.pallas import tpu as pltpu

_PATCH = 3
_PAD = _PATCH // 2
_NUM_PAIRS = 4


def _loss_kernel(f8_ref, gm_ref, x_ref, y_ref, out_ref, *, H, W, TC, CSUB,
                 n_steps):
    step = pl.program_id(0) * pl.num_programs(1) + pl.program_id(1)

    @pl.when(step == 0)
    def _():
        out_ref[...] = jnp.zeros_like(out_ref)

    K = CSUB * H
    f8 = f8_ref[...]

    def chunk_Z(xc3, yc3, Z):
        xc = xc3.reshape(K, W)
        yc = yc3.reshape(K, W)
        rp = lambda a: pltpu.roll(a, W - 1, axis=1)
        rm = lambda a: pltpu.roll(a, 1, axis=1)

        XCh = jnp.concatenate([xc, xc], axis=1).astype(jnp.bfloat16)
        YCh = jnp.concatenate([yc, yc], axis=1).astype(jnp.bfloat16)

        def pair_duo(pxA, pyA, pxB, pyB, idx, Z):
            one = jnp.bfloat16(1.0)
            DX = jnp.concatenate([pxA, pxB], axis=1).astype(jnp.bfloat16) - XCh
            DY = jnp.concatenate([pyA, pyB], axis=1).astype(jnp.bfloat16) - YCh
            NX = DX * jax.lax.rsqrt(DX * DX + jnp.bfloat16(0.81))
            NY = DY * jax.lax.rsqrt(DY * DY + jnp.bfloat16(0.81))
            D = NX - NY
            R = one / (D * D + jnp.bfloat16(0.1))
            u = jnp.dot(f8, R, preferred_element_type=jnp.float32)
            return Z + u * gm_ref[idx]

        xd = pltpu.roll(xc, K - 1, axis=0)
        yd = pltpu.roll(yc, K - 1, axis=0)
        Z = pair_duo(rp(xc), rp(yc), xd, yd, 0, Z)
        Z = pair_duo(rm(xd), rm(yd), rp(xd), rp(yd), 1, Z)
        return Z

    n_chunks = TC // CSUB
    Z0 = jnp.zeros((8, 2 * W), jnp.float32)

    if n_chunks <= 4:
        Z = Z0
        for ci in range(n_chunks):
            Z = chunk_Z(x_ref[0, pl.ds(ci * CSUB, CSUB), :, :],
                        y_ref[0, pl.ds(ci * CSUB, CSUB), :, :], Z)
    else:
        def body(ci, Z):
            c0 = pl.multiple_of(ci * CSUB, CSUB)
            return chunk_Z(x_ref[0, pl.ds(c0, CSUB), :, :],
                           y_ref[0, pl.ds(c0, CSUB), :, :], Z)

        Z = jax.lax.fori_loop(0, n_chunks, body, Z0)
    out_ref[...] += jnp.sum(Z).reshape(1, 1)


@jax.jit
def _geometry_loss(x, y):
    B, C, H, W = x.shape
    HW = H * W

    xf = x.astype(jnp.float32)
    yf = y.astype(jnp.float32)

    TC = C
    for cand in (64, 32, C):
        if C % cand == 0:
            TC = cand
            break
    n_ct = C // TC
    CSUB = 32 if TC % 32 == 0 else (8 if TC % 8 == 0 else TC)

    f = ((np.arange(H) >= _PAD) & (np.arange(H) < H - _PAD)).astype(np.float32)
    g = ((np.arange(W) >= _PAD) & (np.arange(W) < W - _PAD)).astype(np.float32)
    f1 = np.concatenate([f[1:], [0.0]]).astype(np.float32)
    g1p = np.roll(g, -1)
    g1m = np.roll(g, 1)

    K = CSUB * H
    F8 = np.zeros((8, K), np.float32)
    F8[0] = np.tile(f, CSUB)
    F8[1] = np.tile(f1, CSUB)

    Gm = np.zeros((2, 8, 2 * W), np.float32)
    Gm[0, 0, :W] = g + g1p
    Gm[0, 0, W:] = g
    Gm[0, 1, W:] = g
    Gm[1, 0, :W] = g
    Gm[1, 1, :W] = g1m
    Gm[1, 0, W:] = g
    Gm[1, 1, W:] = g1p

    pair_offsets = (1, W - 1, W, W + 1)
    interior = (f[:, None] * g[None, :]).reshape(HW)
    mask_sum = sum(float((interior + np.roll(interior, -s)).sum())
                   for s in pair_offsets)
    mask_const = mask_sum * B * C
    inv_norm = 1.0 / (C * _PATCH * _PATCH * B * H * W)

    kernel_body = functools.partial(
        _loss_kernel, H=H, W=W, TC=TC, CSUB=CSUB, n_steps=B * n_ct)

    out = pl.pallas_call(
        kernel_body,
        out_shape=jax.ShapeDtypeStruct((1, 1), jnp.float32),
        grid=(B, n_ct),
        in_specs=[
            pl.BlockSpec((8, K), lambda b, c: (0, 0)),
            pl.BlockSpec((2, 8, 2 * W), lambda b, c: (0, 0, 0)),
            pl.BlockSpec((1, TC, H, W), lambda b, c: (b, c, 0, 0)),
            pl.BlockSpec((1, TC, H, W), lambda b, c: (b, c, 0, 0)),
        ],
        out_specs=pl.BlockSpec((1, 1), lambda b, c: (0, 0)),
        compiler_params=pltpu.CompilerParams(
            dimension_semantics=("arbitrary", "arbitrary"),
            vmem_limit_bytes=int(40 << 20)),
    )(jnp.asarray(F8, jnp.bfloat16), jnp.asarray(Gm), xf, yf)

    return (mask_const - 0.1 * out[0, 0]) * inv_norm


def kernel(x, y):
    return _geometry_loss(x, y)


# speedup vs baseline: 1.2315x; 1.0061x over previous
import functools

import numpy as np

import jax
import jax.numpy as jnp
from jax.experimental import pallas as pl
from jax.experimental.pallas import tpu as pltpu

_PATCH = 3
_PAD = _PATCH // 2
_NUM_PAIRS = 4


def _loss_kernel(f8_ref, gm_ref, x_ref, y_ref, out_ref, *, H, W, TC, CSUB,
                 n_steps):
    step = pl.program_id(0) * pl.num_programs(1) + pl.program_id(1)

    @pl.when(step == 0)
    def _():
        out_ref[...] = jnp.zeros_like(out_ref)

    K = CSUB * H
    f8 = f8_ref[...]

    def chunk_Z(xc3, yc3, Z):
        xc = xc3.reshape(K, W)
        yc = yc3.reshape(K, W)
        rp = lambda a: pltpu.roll(a, W - 1, axis=1)
        rm = lambda a: pltpu.roll(a, 1, axis=1)

        XCh = jnp.concatenate([xc, xc], axis=1).astype(jnp.bfloat16)
        YCh = jnp.concatenate([yc, yc], axis=1).astype(jnp.bfloat16)

        def pair_duo(pxA, pyA, pxB, pyB, idx, Z):
            one = jnp.bfloat16(1.0)
            DX = jnp.concatenate([pxA, pxB], axis=1).astype(jnp.bfloat16) - XCh
            DY = jnp.concatenate([pyA, pyB], axis=1).astype(jnp.bfloat16) - YCh
            NX = DX * jax.lax.rsqrt(DX * DX + jnp.bfloat16(0.81))
            NY = DY * jax.lax.rsqrt(DY * DY + jnp.bfloat16(0.81))
            D = NX - NY
            R = one / (D * D + jnp.bfloat16(0.1))
            u = jnp.dot(f8, R.astype(jnp.float32),
                        preferred_element_type=jnp.float32)
            return Z + u * gm_ref[idx]

        xd = pltpu.roll(xc, K - 1, axis=0)
        yd = pltpu.roll(yc, K - 1, axis=0)
        Z = pair_duo(rp(xc), rp(yc), xd, yd, 0, Z)
        Z = pair_duo(rm(xd), rm(yd), rp(xd), rp(yd), 1, Z)
        return Z

    n_chunks = TC // CSUB
    Z0 = jnp.zeros((8, 2 * W), jnp.float32)

    if n_chunks <= 4:
        Z = Z0
        for ci in range(n_chunks):
            Z = chunk_Z(x_ref[0, pl.ds(ci * CSUB, CSUB), :, :],
                        y_ref[0, pl.ds(ci * CSUB, CSUB), :, :], Z)
    else:
        def body(ci, Z):
            c0 = pl.multiple_of(ci * CSUB, CSUB)
            return chunk_Z(x_ref[0, pl.ds(c0, CSUB), :, :],
                           y_ref[0, pl.ds(c0, CSUB), :, :], Z)

        Z = jax.lax.fori_loop(0, n_chunks, body, Z0)
    out_ref[...] += jnp.sum(Z).reshape(1, 1)


@jax.jit
def _geometry_loss(x, y):
    B, C, H, W = x.shape
    HW = H * W

    xf = x.astype(jnp.float32)
    yf = y.astype(jnp.float32)

    TC = C
    for cand in (64, 32, C):
        if C % cand == 0:
            TC = cand
            break
    n_ct = C // TC
    CSUB = 32 if TC % 32 == 0 else (8 if TC % 8 == 0 else TC)

    f = ((np.arange(H) >= _PAD) & (np.arange(H) < H - _PAD)).astype(np.float32)
    g = ((np.arange(W) >= _PAD) & (np.arange(W) < W - _PAD)).astype(np.float32)
    f1 = np.concatenate([f[1:], [0.0]]).astype(np.float32)
    g1p = np.roll(g, -1)
    g1m = np.roll(g, 1)

    K = CSUB * H
    F8 = np.zeros((8, K), np.float32)
    F8[0] = np.tile(f, CSUB)
    F8[1] = np.tile(f1, CSUB)

    Gm = np.zeros((2, 8, 2 * W), np.float32)
    Gm[0, 0, :W] = g + g1p
    Gm[0, 0, W:] = g
    Gm[0, 1, W:] = g
    Gm[1, 0, :W] = g
    Gm[1, 1, :W] = g1m
    Gm[1, 0, W:] = g
    Gm[1, 1, W:] = g1p

    pair_offsets = (1, W - 1, W, W + 1)
    interior = (f[:, None] * g[None, :]).reshape(HW)
    mask_sum = sum(float((interior + np.roll(interior, -s)).sum())
                   for s in pair_offsets)
    mask_const = mask_sum * B * C
    inv_norm = 1.0 / (C * _PATCH * _PATCH * B * H * W)

    kernel_body = functools.partial(
        _loss_kernel, H=H, W=W, TC=TC, CSUB=CSUB, n_steps=B * n_ct)

    out = pl.pallas_call(
        kernel_body,
        out_shape=jax.ShapeDtypeStruct((1, 1), jnp.float32),
        grid=(B, n_ct),
        in_specs=[
            pl.BlockSpec((8, K), lambda b, c: (0, 0)),
            pl.BlockSpec((2, 8, 2 * W), lambda b, c: (0, 0, 0)),
            pl.BlockSpec((1, TC, H, W), lambda b, c: (b, c, 0, 0)),
            pl.BlockSpec((1, TC, H, W), lambda b, c: (b, c, 0, 0)),
        ],
        out_specs=pl.BlockSpec((1, 1), lambda b, c: (0, 0)),
        compiler_params=pltpu.CompilerParams(
            dimension_semantics=("arbitrary", "arbitrary"),
            vmem_limit_bytes=int(40 << 20)),
    )(jnp.asarray(F8), jnp.asarray(Gm), xf, yf)

    return (mask_const - 0.1 * out[0, 0]) * inv_norm


def kernel(x, y):
    return _geometry_loss(x, y)
